# baseline (speedup 1.0000x reference)
import jax
import jax.numpy as jnp
from jax import lax
from jax.experimental import pallas as pl
from jax.experimental.pallas import tpu as pltpu
from jax.experimental.pallas import tpu_sc as plsc

N_NODES = 100000
NP = 100096
N_EDGES = 3200000
NU = 6
PW = 8
RB = 100
ROWS = N_EDGES // RB
NC = 2
NS = 16
NW = NC * NS
RW = ROWS // NW
K = 8
CHUNK_E = K * RB
ITERS = RW // K
GROUPS = CHUNK_E // 16
NPT = NP // NS


def _uke_body(x_ref, w_ref, o_ref):
    x = x_ref[...]
    w = w_ref[0]
    col = lax.broadcasted_iota(jnp.int32, x.shape, 1)
    valid = col < NU
    m = jnp.max(jnp.where(valid, x, -jnp.inf), axis=1, keepdims=True)
    e = jnp.where(valid, jnp.exp(x - m), 0.0)
    s = jnp.sum(e, axis=1, keepdims=True)
    o_ref[...] = x + w * (e / s)


BR = 6256


def _unary_ke(unary_pad, unary_cw):
    return pl.pallas_call(
        _uke_body,
        grid=(NP // BR,),
        out_shape=jax.ShapeDtypeStruct((NP, PW), jnp.float32),
        in_specs=[
            pl.BlockSpec((BR, PW), lambda i: (i, 0)),
            pl.BlockSpec(memory_space=pltpu.SMEM),
        ],
        out_specs=pl.BlockSpec((BR, PW), lambda i: (i, 0)),
    )(unary_pad, unary_cw)


def _combine_body(u_ref, p_ref, o_ref):
    o_ref[...] = u_ref[...] + p_ref[0] + p_ref[1]


def _combine(u_pad, partial):
    uf = u_pad.reshape(NP * PW // 128, 128)
    pf = partial.reshape(NC, NP * PW // 128, 128)
    out = pl.pallas_call(
        _combine_body,
        out_shape=jax.ShapeDtypeStruct((NP * PW // 128, 128), jnp.float32),
    )(uf, pf)
    return out.reshape(NP, PW)


def _edge_body(u_hbm, i1_hbm, i2_hbm, b_hbm, cw_hbm, zero_hbm,
               part_hbm, bp_hbm,
               acc,
               i1b0, i1b1, i2b0, i2b1,
               bb0, bb1, bpb0, bpb1,
               u1b0, u1b1, u2b0, u2b1,
               dxb0, dxb1, dyb0, dyb1,
               cwb, gsem, ssem, bsem):
    c = lax.axis_index("c")
    s_id = lax.axis_index("s")
    wid = s_id * NC + c
    iota = lax.iota(jnp.int32, 16)

    i1b = (i1b0, i1b1)
    i2b = (i2b0, i2b1)
    bb = (bb0, bb1)
    bpb = (bpb0, bpb1)
    u1b = (u1b0, u1b1)
    u2b = (u2b0, u2b1)
    dxb = (dxb0, dxb1)
    dyb = (dyb0, dyb1)

    pltpu.sync_copy(cw_hbm, cwb)
    pltpu.sync_copy(zero_hbm, acc.at[pl.ds(s_id * NPT, NPT)])
    plsc.subcore_barrier()

    cwv = cwb[pl.ds(0, 16)]
    w = [cwv[i] for i in range(NU)]
    zv = jnp.zeros((16,), jnp.float32)

    def _zero_pad(g, carry):
        r_idx = g * 16 + iota
        for ci in (NU, NU + 1):
            cv = jnp.full((16,), ci, jnp.int32)
            for buf in (dxb0, dxb1, dyb0, dyb1):
                plsc.store_scatter(buf, [r_idx, cv], zv)
        return carry
    lax.fori_loop(0, GROUPS, _zero_pad, 0)

    def _prefetch(it, p):
        r0 = wid * RW + it * K
        e0 = wid * (RW * RB) + it * CHUNK_E
        pltpu.sync_copy(i1_hbm.at[pl.ds(r0, K)], i1b[p])
        pltpu.sync_copy(i2_hbm.at[pl.ds(r0, K)], i2b[p])
        pltpu.sync_copy(b_hbm.at[pl.ds(e0, CHUNK_E)], bb[p])

    def _fire_gathers(p):
        for j in range(K):
            pltpu.async_copy(
                u_hbm.at[i1b[p].at[j]], u1b[p].at[pl.ds(j * RB, RB)], gsem)
            pltpu.async_copy(
                u_hbm.at[i2b[p].at[j]], u2b[p].at[pl.ds(j * RB, RB)], gsem)

    def _drain_gathers(p):
        for j in range(K):
            pltpu.make_async_copy(
                u_hbm.at[i1b[p].at[j]], u1b[p].at[pl.ds(j * RB, RB)],
                gsem).wait()
            pltpu.make_async_copy(
                u_hbm.at[i2b[p].at[j]], u2b[p].at[pl.ds(j * RB, RB)],
                gsem).wait()

    def _fire_scatters(p):
        for j in range(K):
            pltpu.async_copy(
                dxb[p].at[pl.ds(j * RB, RB)], acc.at[i1b[p].at[j]],
                ssem, add=True)
            pltpu.async_copy(
                dyb[p].at[pl.ds(j * RB, RB)], acc.at[i2b[p].at[j]],
                ssem, add=True)

    def _drain_scatters(p):
        for j in range(K):
            pltpu.make_async_copy(
                dxb[p].at[pl.ds(j * RB, RB)], acc.at[i1b[p].at[j]],
                ssem).wait()
            pltpu.make_async_copy(
                dyb[p].at[pl.ds(j * RB, RB)], acc.at[i2b[p].at[j]],
                ssem).wait()

    def _compute(it, p):
        e0 = wid * (RW * RB) + it * CHUNK_E

        @plsc.parallel_loop(0, GROUPS, 1, unroll=4)
        def _group(g):
            base = g * 16
            r_idx = base + iota
            bv = bb[p][pl.ds(base, 16)]
            ebn = jnp.exp(-bv)
            db = jnp.zeros((16,), jnp.float32)
            for i in range(NU):
                cv = jnp.full((16,), i, jnp.int32)
                u1v = plsc.load_gather(u1b[p], [r_idx, cv])
                u2v = plsc.load_gather(u2b[p], [r_idx, cv])
                ea = jnp.exp(-u1v)
                ec = jnp.exp(u2v)
                t = w[i] / (ea + ebn + ec)
                plsc.store_scatter(dxb[p], [r_idx, cv], -(ea * t))
                plsc.store_scatter(dyb[p], [r_idx, cv], ec * t)
                db = db - ebn * t
            bpb[p][pl.ds(base, 16)] = bv + db

        pltpu.async_copy(bpb[p], bp_hbm.at[pl.ds(e0, CHUNK_E)], bsem)

    def _drain_bp(it, p):
        e0 = wid * (RW * RB) + it * CHUNK_E
        pltpu.make_async_copy(
            bpb[p], bp_hbm.at[pl.ds(e0, CHUNK_E)], bsem).wait()

    def _pipeline_step(it, p):
        q = 1 - p
        _drain_gathers(p)

        @pl.when(it >= 2)
        def _():
            _drain_bp(it - 2, p)

        @pl.when(it >= 1)
        def _():
            _drain_scatters(q)

        @pl.when(it + 1 < ITERS)
        def _():
            _prefetch(it + 1, q)
            _fire_gathers(q)

        _compute(it, p)

        _fire_scatters(p)

    _prefetch(0, 0)
    _fire_gathers(0)

    def _outer(o, carry):
        _pipeline_step(2 * o, 0)
        _pipeline_step(2 * o + 1, 1)
        return carry
    lax.fori_loop(0, (ITERS - 1) // 2, _outer, 0)
    _pipeline_step(ITERS - 1, 0)

    _drain_scatters(0)
    _drain_bp(ITERS - 2, 1)
    _drain_bp(ITERS - 1, 0)

    plsc.subcore_barrier()
    pltpu.sync_copy(acc.at[pl.ds(s_id * NPT, NPT)],
                    part_hbm.at[c, pl.ds(s_id * NPT, NPT)])


def _edge_call(u_pad, i1, i2, bflat, cw16, zeros_hbm):
    mesh = plsc.VectorSubcoreMesh(
        core_axis_name="c", subcore_axis_name="s",
        num_cores=NC, num_subcores=NS)
    return pl.kernel(
        _edge_body,
        compiler_params=pltpu.CompilerParams(
            needs_layout_passes=False, use_tc_tiling_on_sc=False),
        out_type=[
            jax.ShapeDtypeStruct((NC, NP, PW), jnp.float32),
            jax.ShapeDtypeStruct((N_EDGES,), jnp.float32),
        ],
        mesh=mesh,
        scratch_types=(
            [pltpu.VMEM_SHARED((NP, PW), jnp.float32)]
            + [pltpu.VMEM((K, RB), jnp.int32)] * 4
            + [pltpu.VMEM((CHUNK_E,), jnp.float32)] * 4
            + [pltpu.VMEM((CHUNK_E, PW), jnp.float32)] * 8
            + [pltpu.VMEM((16,), jnp.float32)]
            + [pltpu.SemaphoreType.DMA,
               pltpu.SemaphoreType.DMA,
               pltpu.SemaphoreType.DMA]
        ),
    )(u_pad, i1, i2, bflat, cw16, zeros_hbm)


@jax.jit
def kernel(unary, binary, index1, index2, unary_cw, binary_cw):
    unary_pad = jnp.pad(unary, ((0, NP - N_NODES), (0, PW - NU)))
    u_pad = _unary_ke(unary_pad, unary_cw)

    i1 = index1.reshape(ROWS, RB)
    i2 = index2.reshape(ROWS, RB)
    bflat = binary.reshape(N_EDGES)
    cw16 = jnp.pad(binary_cw, (0, 16 - NU))
    zeros_hbm = jnp.zeros((NPT, PW), jnp.float32)

    partial, bp = _edge_call(u_pad, i1, i2, bflat, cw16, zeros_hbm)
    up_pad = _combine(u_pad, partial)
    return up_pad[:N_NODES, :NU], bp.reshape(N_EDGES, 1)


# speedup vs baseline: 1.5857x; 1.5857x over previous
import jax
import jax.numpy as jnp
from jax import lax
from jax.experimental import pallas as pl
from jax.experimental.pallas import tpu as pltpu
from jax.experimental.pallas import tpu_sc as plsc

N_NODES = 100000
NP = 100096
N_EDGES = 3200000
NU = 6
PW = 8
RB = 100
ROWS = N_EDGES // RB
NC = 2
NS = 16
NW = NC * NS
RW = ROWS // NW
K = 8
CHUNK_E = K * RB
ITERS = RW // K
GROUPS = CHUNK_E // 16
NPT = NP // NS


def _uke_body(x_ref, w_ref, o_ref):
    x = x_ref[...]
    w = w_ref[0]
    col = lax.broadcasted_iota(jnp.int32, x.shape, 1)
    valid = col < NU
    m = jnp.max(jnp.where(valid, x, -jnp.inf), axis=1, keepdims=True)
    e = jnp.where(valid, jnp.exp(x - m), 0.0)
    s = jnp.sum(e, axis=1, keepdims=True)
    o_ref[...] = x + w * (e / s)


BR = 6256


def _unary_ke(unary_pad, unary_cw):
    return pl.pallas_call(
        _uke_body,
        grid=(NP // BR,),
        out_shape=jax.ShapeDtypeStruct((NP, PW), jnp.float32),
        in_specs=[
            pl.BlockSpec((BR, PW), lambda i: (i, 0)),
            pl.BlockSpec(memory_space=pltpu.SMEM),
        ],
        out_specs=pl.BlockSpec((BR, PW), lambda i: (i, 0)),
    )(unary_pad, unary_cw)


def _combine_body(u_ref, p_ref, o_ref):
    o_ref[...] = u_ref[...] + p_ref[0] + p_ref[1]


def _combine(u_pad, partial):
    uf = u_pad.reshape(NP * PW // 128, 128)
    pf = partial.reshape(NC, NP * PW // 128, 128)
    out = pl.pallas_call(
        _combine_body,
        out_shape=jax.ShapeDtypeStruct((NP * PW // 128, 128), jnp.float32),
    )(uf, pf)
    return out.reshape(NP, PW)


def _edge_body(u_hbm, i1_hbm, i2_hbm, b_hbm, cw_hbm, zero_hbm,
               part_hbm, bp_hbm,
               acc,
               i1b0, i1b1, i2b0, i2b1,
               bb0, bb1, bpb0, bpb1,
               u1b0, u1b1, u2b0, u2b1,
               dxb0, dxb1, dyb0, dyb1,
               cwb, gsem, ssem, bsem):
    c = lax.axis_index("c")
    s_id = lax.axis_index("s")
    wid = s_id * NC + c
    iota = lax.iota(jnp.int32, 16)

    i1b = (i1b0, i1b1)
    i2b = (i2b0, i2b1)
    bb = (bb0, bb1)
    bpb = (bpb0, bpb1)
    u1b = (u1b0, u1b1)
    u2b = (u2b0, u2b1)
    dxb = (dxb0, dxb1)
    dyb = (dyb0, dyb1)

    pltpu.sync_copy(cw_hbm, cwb)
    pltpu.sync_copy(zero_hbm, acc.at[pl.ds(s_id * NPT, NPT)])
    plsc.subcore_barrier()

    cwv = cwb[pl.ds(0, 16)]
    w = [cwv[i] for i in range(NU)]
    zv = jnp.zeros((16,), jnp.float32)

    def _zero_pad(g, carry):
        r_idx = g * 16 + iota
        for ci in (NU, NU + 1):
            cv = jnp.full((16,), ci, jnp.int32)
            for buf in (dxb0, dxb1, dyb0, dyb1):
                plsc.store_scatter(buf, [r_idx, cv], zv)
        return carry
    lax.fori_loop(0, GROUPS, _zero_pad, 0)

    def _prefetch(it, p):
        r0 = wid * RW + it * K
        e0 = wid * (RW * RB) + it * CHUNK_E
        pltpu.sync_copy(i1_hbm.at[pl.ds(r0, K)], i1b[p])
        pltpu.sync_copy(i2_hbm.at[pl.ds(r0, K)], i2b[p])
        pltpu.sync_copy(b_hbm.at[pl.ds(e0, CHUNK_E)], bb[p])

    def _fire_gathers(p):
        for j in range(K):
            pltpu.async_copy(
                u_hbm.at[i1b[p].at[j]], u1b[p].at[pl.ds(j * RB, RB)], gsem)
            pltpu.async_copy(
                u_hbm.at[i2b[p].at[j]], u2b[p].at[pl.ds(j * RB, RB)], gsem)

    def _drain_gathers(p):
        for j in range(K):
            pltpu.make_async_copy(
                u_hbm.at[i1b[p].at[j]], u1b[p].at[pl.ds(j * RB, RB)],
                gsem).wait()
            pltpu.make_async_copy(
                u_hbm.at[i2b[p].at[j]], u2b[p].at[pl.ds(j * RB, RB)],
                gsem).wait()

    def _fire_scatters(p):
        for j in range(K):
            pltpu.async_copy(
                dxb[p].at[pl.ds(j * RB, RB)], acc.at[i1b[p].at[j]],
                ssem, add=True)
            pltpu.async_copy(
                dyb[p].at[pl.ds(j * RB, RB)], acc.at[i2b[p].at[j]],
                ssem, add=True)

    def _drain_scatters(p):
        for j in range(K):
            pltpu.make_async_copy(
                dxb[p].at[pl.ds(j * RB, RB)], acc.at[i1b[p].at[j]],
                ssem).wait()
            pltpu.make_async_copy(
                dyb[p].at[pl.ds(j * RB, RB)], acc.at[i2b[p].at[j]],
                ssem).wait()

    def _compute(it, p):
        e0 = wid * (RW * RB) + it * CHUNK_E

        @plsc.parallel_loop(0, GROUPS, 1, unroll=3)
        def _group(g):
            base = g * 16
            r_idx = base + iota
            bv = bb[p][pl.ds(base, 16)]
            ebn = jnp.exp(-bv)
            db = jnp.zeros((16,), jnp.float32)
            for i in range(NU):
                cv = jnp.full((16,), i, jnp.int32)
                u1v = plsc.load_gather(u1b[p], [r_idx, cv])
                u2v = plsc.load_gather(u2b[p], [r_idx, cv])
                ea = jnp.exp(-u1v)
                ec = jnp.exp(u2v)
                t = w[i] / (ea + ebn + ec)
                plsc.store_scatter(dxb[p], [r_idx, cv], -(ea * t))
                plsc.store_scatter(dyb[p], [r_idx, cv], ec * t)
                db = db - ebn * t
            bpb[p][pl.ds(base, 16)] = bv + db

        pltpu.async_copy(bpb[p], bp_hbm.at[pl.ds(e0, CHUNK_E)], bsem)

    def _drain_bp(it, p):
        e0 = wid * (RW * RB) + it * CHUNK_E
        pltpu.make_async_copy(
            bpb[p], bp_hbm.at[pl.ds(e0, CHUNK_E)], bsem).wait()

    def _pipeline_step(it, p):
        q = 1 - p
        _drain_gathers(p)

        @pl.when(it >= 2)
        def _():
            _drain_bp(it - 2, p)

        @pl.when(it >= 1)
        def _():
            _drain_scatters(q)

        @pl.when(it + 1 < ITERS)
        def _():
            _prefetch(it + 1, q)
            _fire_gathers(q)

        _compute(it, p)

        _fire_scatters(p)

    _prefetch(0, 0)
    _fire_gathers(0)

    def _outer(o, carry):
        _pipeline_step(2 * o, 0)
        _pipeline_step(2 * o + 1, 1)
        return carry
    lax.fori_loop(0, (ITERS - 1) // 2, _outer, 0)
    _pipeline_step(ITERS - 1, 0)

    _drain_scatters(0)
    _drain_bp(ITERS - 2, 1)
    _drain_bp(ITERS - 1, 0)

    plsc.subcore_barrier()
    pltpu.sync_copy(acc.at[pl.ds(s_id * NPT, NPT)],
                    part_hbm.at[c, pl.ds(s_id * NPT, NPT)])


def _edge_call(u_pad, i1, i2, bflat, cw16, zeros_hbm):
    mesh = plsc.VectorSubcoreMesh(
        core_axis_name="c", subcore_axis_name="s",
        num_cores=NC, num_subcores=NS)
    return pl.kernel(
        _edge_body,
        compiler_params=pltpu.CompilerParams(
            needs_layout_passes=False, use_tc_tiling_on_sc=False),
        out_type=[
            jax.ShapeDtypeStruct((NC, NP, PW), jnp.float32),
            jax.ShapeDtypeStruct((N_EDGES,), jnp.float32),
        ],
        mesh=mesh,
        scratch_types=(
            [pltpu.VMEM_SHARED((NP, PW), jnp.float32)]
            + [pltpu.VMEM((K, RB), jnp.int32)] * 4
            + [pltpu.VMEM((CHUNK_E,), jnp.float32)] * 4
            + [pltpu.VMEM((CHUNK_E, PW), jnp.float32)] * 8
            + [pltpu.VMEM((16,), jnp.float32)]
            + [pltpu.SemaphoreType.DMA,
               pltpu.SemaphoreType.DMA,
               pltpu.SemaphoreType.DMA]
        ),
    )(u_pad, i1, i2, bflat, cw16, zeros_hbm)


@jax.jit
def kernel(unary, binary, index1, index2, unary_cw, binary_cw):
    unary_pad = jnp.pad(unary, ((0, NP - N_NODES), (0, PW - NU)))
    u_pad = _unary_ke(unary_pad, unary_cw)

    i1 = index1.reshape(ROWS, RB)
    i2 = index2.reshape(ROWS, RB)
    bflat = binary.reshape(N_EDGES)
    cw16 = jnp.pad(binary_cw, (0, 16 - NU))
    zeros_hbm = jnp.zeros((NPT, PW), jnp.float32)

    partial, bp = _edge_call(u_pad, i1, i2, bflat, cw16, zeros_hbm)
    up_pad = _combine(u_pad, partial)
    return up_pad[:N_NODES, :NU], bp.reshape(N_EDGES, 1)
